# Initial kernel scaffold; baseline (speedup 1.0000x reference)
import jax
import jax.numpy as jnp
from jax.experimental import pallas as pl


def kernel(x, Wr, br, W, b):
    raise NotImplementedError("write your pallas kernel here")


# speedup vs baseline: 3.4688x; 3.4688x over previous
import functools

import jax
import jax.numpy as jnp
from jax.experimental import pallas as pl
from jax.experimental.pallas import tpu as pltpu

N, D_IN, D_OUT, E, TOP_K = 4096, 1024, 1024, 8, 2
TT = 512
LANES = 128
NEG = -1e30


def _moe_body(x_ref, wr_ref, br_ref, w_ref, b_ref, out_ref, wfull_ref):
    e = pl.program_id(1)
    lane = jax.lax.broadcasted_iota(jnp.int32, (TT, LANES), 1)

    @pl.when(e == 0)
    def _router():
        logits = jnp.dot(x_ref[...], wr_ref[...],
                         preferred_element_type=jnp.float32) + br_ref[...]
        logits = jnp.where(lane < E, logits, NEG)
        m1 = jnp.max(logits, axis=1, keepdims=True)
        i1 = jnp.min(jnp.where(logits == m1, lane, LANES), axis=1,
                     keepdims=True)
        l2 = jnp.where(lane == i1, NEG, logits)
        m2 = jnp.max(l2, axis=1, keepdims=True)
        i2 = jnp.min(jnp.where(l2 == m2, lane, LANES), axis=1, keepdims=True)
        w0 = 1.0 / (1.0 + jnp.exp(m2 - m1))
        w1 = 1.0 - w0
        wfull_ref[...] = jnp.where(lane == i1, w0, 0.0) + jnp.where(
            lane == i2, w1, 0.0)

    w_e = jnp.sum(jnp.where(lane == e, wfull_ref[...], 0.0), axis=1,
                  keepdims=True)
    z = jnp.dot(x_ref[...], w_ref[0],
                preferred_element_type=jnp.float32) + b_ref[0]
    y = w_e * (0.5 * z * (1.0 + jax.lax.erf(z * 0.7071067811865476)))

    @pl.when(e == 0)
    def _init():
        out_ref[...] = y

    @pl.when(e != 0)
    def _acc():
        out_ref[...] += y


@jax.jit
def kernel(x, Wr, br, W, b):
    wr_pad = jnp.zeros((D_IN, LANES), jnp.float32).at[:, :E].set(Wr)
    br_pad = jnp.zeros((1, LANES), jnp.float32).at[0, :E].set(br)
    grid = (N // TT, E)
    return pl.pallas_call(
        _moe_body,
        grid=grid,
        in_specs=[
            pl.BlockSpec((TT, D_IN), lambda t, e: (t, 0)),
            pl.BlockSpec((D_IN, LANES), lambda t, e: (0, 0)),
            pl.BlockSpec((1, LANES), lambda t, e: (0, 0)),
            pl.BlockSpec((1, D_IN, D_OUT), lambda t, e: (e, 0, 0)),
            pl.BlockSpec((1, 1, D_OUT), lambda t, e: (e, 0, 0)),
        ],
        out_specs=pl.BlockSpec((TT, D_OUT), lambda t, e: (t, 0)),
        out_shape=jax.ShapeDtypeStruct((N, D_OUT), jnp.float32),
        scratch_shapes=[pltpu.VMEM((TT, LANES), jnp.float32)],
        compiler_params=pltpu.CompilerParams(
            dimension_semantics=("arbitrary", "arbitrary"),
        ),
    )(x, wr_pad, br_pad, W, b.reshape(E, 1, D_OUT))


# speedup vs baseline: 4.3484x; 1.2536x over previous
import functools

import jax
import jax.numpy as jnp
from jax.experimental import pallas as pl
from jax.experimental.pallas import tpu as pltpu

N, D_IN, D_OUT, E, TOP_K = 4096, 1024, 1024, 8, 2
TT = 2048
LANES = 128
NEG = -1e30


def _moe_body(x_ref, wr_ref, br_ref, w_ref, b_ref, out_ref, wfull_ref):
    e = pl.program_id(1)
    lane = jax.lax.broadcasted_iota(jnp.int32, (TT, LANES), 1)

    @pl.when(e == 0)
    def _router():
        logits = jnp.dot(x_ref[...], wr_ref[...],
                         preferred_element_type=jnp.float32) + br_ref[...]
        logits = jnp.where(lane < E, logits, NEG)
        m1 = jnp.max(logits, axis=1, keepdims=True)
        i1 = jnp.min(jnp.where(logits == m1, lane, LANES), axis=1,
                     keepdims=True)
        l2 = jnp.where(lane == i1, NEG, logits)
        m2 = jnp.max(l2, axis=1, keepdims=True)
        i2 = jnp.min(jnp.where(l2 == m2, lane, LANES), axis=1, keepdims=True)
        w0 = 1.0 / (1.0 + jnp.exp(m2 - m1))
        w1 = 1.0 - w0
        wfull_ref[...] = jnp.where(lane == i1, w0, 0.0) + jnp.where(
            lane == i2, w1, 0.0)

    w_e = jnp.sum(jnp.where(lane == e, wfull_ref[...], 0.0), axis=1,
                  keepdims=True)
    z = jnp.dot(x_ref[...], w_ref[0],
                preferred_element_type=jnp.float32) + b_ref[0]
    y = w_e * (0.5 * z * (1.0 + jax.lax.erf(z * 0.7071067811865476)))

    @pl.when(e == 0)
    def _init():
        out_ref[...] = y

    @pl.when(e != 0)
    def _acc():
        out_ref[...] += y


@jax.jit
def kernel(x, Wr, br, W, b):
    wr_pad = jnp.zeros((D_IN, LANES), jnp.float32).at[:, :E].set(Wr)
    br_pad = jnp.zeros((1, LANES), jnp.float32).at[0, :E].set(br)
    grid = (N // TT, E)
    return pl.pallas_call(
        _moe_body,
        grid=grid,
        in_specs=[
            pl.BlockSpec((TT, D_IN), lambda t, e: (t, 0)),
            pl.BlockSpec((D_IN, LANES), lambda t, e: (0, 0)),
            pl.BlockSpec((1, LANES), lambda t, e: (0, 0)),
            pl.BlockSpec((1, D_IN, D_OUT), lambda t, e: (e, 0, 0)),
            pl.BlockSpec((1, 1, D_OUT), lambda t, e: (e, 0, 0)),
        ],
        out_specs=pl.BlockSpec((TT, D_OUT), lambda t, e: (t, 0)),
        out_shape=jax.ShapeDtypeStruct((N, D_OUT), jnp.float32),
        scratch_shapes=[pltpu.VMEM((TT, LANES), jnp.float32)],
        compiler_params=pltpu.CompilerParams(
            dimension_semantics=("arbitrary", "arbitrary"),
        ),
    )(x, wr_pad, br_pad, W, b.reshape(E, 1, D_OUT))
